# Initial kernel scaffold; baseline (speedup 1.0000x reference)
import jax
import jax.numpy as jnp
from jax.experimental import pallas as pl


def kernel(node_attr, edge_attr, node_mask, edge_mask, batching, params):
    raise NotImplementedError("write your pallas kernel here")


# speedup vs baseline: 28.2250x; 28.2250x over previous
import functools

import jax
import jax.numpy as jnp
from jax.experimental import pallas as pl
from jax.experimental.pallas import tpu as pltpu

B, N = 4, 64
D_NODE = 64
D_EDGE = 16
CONV = [64, 64]
FC = [128, 10]
NN = N * N


def _fused_kernel(eT_ref, x_ref, emT_ref, em2_ref, nm_ref,
                  w00_ref, b00_ref, w01_ref, b01_ref, w02f_ref, b02r_ref,
                  r0w_ref, r0b_ref,
                  w10_ref, b10_ref, w11_ref, b11_ref, w12f_ref, b12r_ref,
                  r1w_ref, r1b_ref,
                  fw0_ref, fb0_ref, fw1_ref, fb1_ref,
                  out_ref):
    e2 = eT_ref[0]
    em = emT_ref[0]
    em2d = em2_ref[0]
    nmv = nm_ref[0]
    x = x_ref[0]

    layers = (
        (w00_ref, b00_ref, w01_ref, b01_ref, w02f_ref, b02r_ref, r0w_ref, r0b_ref),
        (w10_ref, b10_ref, w11_ref, b11_ref, w12f_ref, b12r_ref, r1w_ref, r1b_ref),
    )

    for (w0, b0, w1, b1, w2f, b2r, rw, rb) in layers:
        h = jnp.maximum(jnp.dot(e2, w0[...], preferred_element_type=jnp.float32)
                        + b0[...], 0.0)
        h = jnp.maximum(jnp.dot(h, w1[...], preferred_element_type=jnp.float32)
                        + b1[...], 0.0)
        h = h * em
        h3 = h.reshape(N, N, CONV[0])
        xb = jnp.broadcast_to(x[None], (N, N, D_NODE))
        c = jax.lax.dot_general(h3, xb, (((1,), (1,)), ((0,), (0,))),
                                preferred_element_type=jnp.float32)
        cf = c.reshape(N, CONV[0] * D_NODE)
        msg = jnp.dot(cf, w2f[...], preferred_element_type=jnp.float32)
        xe = jnp.dot(em2d, x, preferred_element_type=jnp.float32)
        msg = msg + jnp.dot(xe, b2r[...], preferred_element_type=jnp.float32)
        z = jnp.dot(x, rw[...], preferred_element_type=jnp.float32) + rb[...] + msg
        x = jnp.maximum(z * nmv, 0.0)

    pooled = jnp.sum(x * nmv, axis=0, keepdims=True)
    o = jnp.maximum(jnp.dot(pooled, fw0_ref[...],
                            preferred_element_type=jnp.float32) + fb0_ref[...], 0.0)
    o = jnp.dot(o, fw1_ref[...], preferred_element_type=jnp.float32) + fb1_ref[...]
    m = jnp.max(o, axis=-1, keepdims=True)
    e = jnp.exp(o - m)
    out_ref[0] = e / jnp.sum(e, axis=-1, keepdims=True)


@functools.partial(jax.jit, static_argnames=("interpret",))
def _run(node_attr, edge_attr, node_mask, edge_mask, params, interpret=False):
    f32 = jnp.float32
    eT = edge_attr.transpose(0, 2, 1, 3).reshape(B, NN, D_EDGE)
    em2 = edge_mask.transpose(0, 2, 1)
    emT = em2.reshape(B, NN, 1)
    nm = node_mask.reshape(B, N, 1)

    ops = [eT, node_attr, emT, em2, nm]
    for l in range(2):
        fin = D_NODE if l == 0 else CONV[l - 1]
        ops += [
            params[f"conv{l}_enet_W0"], params[f"conv{l}_enet_b0"].reshape(1, -1),
            params[f"conv{l}_enet_W1"], params[f"conv{l}_enet_b1"].reshape(1, -1),
            params[f"conv{l}_enet_W2"].reshape(-1, CONV[l]),
            params[f"conv{l}_enet_b2"].reshape(fin, CONV[l]),
            params[f"conv{l}_root_W"], params[f"conv{l}_root_b"].reshape(1, -1),
        ]
    ops += [
        params["fc_W0"], params["fc_b0"].reshape(1, -1),
        params["fc_W1"], params["fc_b1"].reshape(1, -1),
    ]
    ops = [o.astype(f32) for o in ops]

    def batch_spec(shape):
        return pl.BlockSpec((1,) + shape[1:], lambda b: (b,) + (0,) * (len(shape) - 1))

    def whole_spec(shape):
        return pl.BlockSpec(shape, lambda b: (0,) * len(shape))

    in_specs = [batch_spec(ops[i].shape) for i in range(5)]
    in_specs += [whole_spec(o.shape) for o in ops[5:]]

    return pl.pallas_call(
        _fused_kernel,
        grid=(B,),
        in_specs=in_specs,
        out_specs=pl.BlockSpec((1, 1, FC[-1]), lambda b: (b, 0, 0)),
        out_shape=jax.ShapeDtypeStruct((B, 1, FC[-1]), f32),
        compiler_params=pltpu.CompilerParams(
            dimension_semantics=("arbitrary",),
        ),
        interpret=interpret,
    )(*ops).reshape(B, FC[-1])


def kernel(node_attr, edge_attr, node_mask, edge_mask, batching, params):
    del batching
    return _run(node_attr, edge_attr, node_mask, edge_mask, params)
